# Initial kernel scaffold; baseline (speedup 1.0000x reference)
import jax
import jax.numpy as jnp
from jax.experimental import pallas as pl


def kernel(x, edge_index, edge_weight, W_in, b_in, W_out, b_out):
    raise NotImplementedError("write your pallas kernel here")


# speedup vs baseline: 3.6226x; 3.6226x over previous
import functools

import jax
import jax.numpy as jnp
from jax import lax
from jax.experimental import pallas as pl
from jax.experimental.pallas import tpu as pltpu
from jax.experimental.pallas import tpu_sc as plsc

ALPHA = 0.1
KHOP = 3
NC = 2
NS = 16
NW = NC * NS
ECHUNK = 128
LANES = 16


def _mlp_in_body(x_ref, w_ref, b_ref, o_ref):
    h = jnp.dot(x_ref[...], w_ref[...], preferred_element_type=jnp.float32)
    o_ref[...] = jnp.maximum(h + b_ref[...], 0.0)


def _norm_body(h0_ref, ho_ref, hi_ref, hs_ref, f0a_ref, nins_ref, nout_ref):
    deg_out = ho_ref[0, :, 0:1] + ho_ref[1, :, 0:1]
    deg_in = hi_ref[0, :, 0:1] + hi_ref[1, :, 0:1]
    no = jnp.where(deg_out > 0, lax.rsqrt(deg_out), 0.0)
    ni = jnp.where(deg_in > 0, lax.rsqrt(deg_in), 0.0)
    h0 = h0_ref[...]
    hs_ref[...] = h0 * no
    f0a_ref[...] = ALPHA * h0
    nins_ref[...] = (1.0 - ALPHA) * ni
    nout_ref[...] = no


def _combine_body(p_ref, nins_ref, nout_ref, f0a_ref, o_ref):
    h = (p_ref[0] + p_ref[1]) * nins_ref[...] + f0a_ref[...]
    o_ref[...] = h * nout_ref[...]


def _final_body(p_ref, nins_ref, f0a_ref, w_ref, b_ref, o_ref):
    h = (p_ref[0] + p_ref[1]) * nins_ref[...] + f0a_ref[...]
    o_ref[...] = jnp.dot(h, w_ref[...], preferred_element_type=jnp.float32) + b_ref[...]


def _tc_mlp_in(x, w, b):
    return pl.pallas_call(
        _mlp_in_body,
        out_shape=jax.ShapeDtypeStruct((x.shape[0], w.shape[1]), jnp.float32),
        name="tc_mlp_in",
    )(x, w, b)


def _tc_norm(h0, ho, hi):
    n, hid = h0.shape
    return pl.pallas_call(
        _norm_body,
        out_shape=[
            jax.ShapeDtypeStruct((n, hid), jnp.float32),
            jax.ShapeDtypeStruct((n, hid), jnp.float32),
            jax.ShapeDtypeStruct((n, 1), jnp.float32),
            jax.ShapeDtypeStruct((n, 1), jnp.float32),
        ],
        name="tc_norm",
    )(h0, ho, hi)


def _tc_combine(p, nins, nout, f0a):
    return pl.pallas_call(
        _combine_body,
        out_shape=jax.ShapeDtypeStruct(f0a.shape, jnp.float32),
        name="tc_combine",
    )(p, nins, nout, f0a)


def _tc_final(p, nins, f0a, w, b):
    n = f0a.shape[0]
    return pl.pallas_call(
        _final_body,
        out_shape=jax.ShapeDtypeStruct((n, w.shape[1]), jnp.float32),
        name="tc_final",
    )(p, nins, f0a, w, b)


def _sc_hist(src, dst, z16):
    n = z16.shape[0]
    e = src.shape[0]
    ew = e // NW
    full = ew // ECHUNK
    tail16 = (ew - full * ECHUNK) // LANES
    rps = n // NS
    mesh = plsc.VectorSubcoreMesh(core_axis_name="c", subcore_axis_name="s")
    out_t = jax.ShapeDtypeStruct((NC, n, LANES), jnp.float32)

    @functools.partial(
        pl.kernel,
        out_type=[out_t, out_t],
        mesh=mesh,
        scratch_types=[
            pltpu.VMEM((ECHUNK,), jnp.int32),
            pltpu.VMEM((ECHUNK,), jnp.int32),
            pltpu.VMEM((LANES,), jnp.int32),
            pltpu.VMEM((LANES,), jnp.int32),
            pltpu.VMEM((ECHUNK, LANES), jnp.float32),
            pltpu.VMEM_SHARED((n, LANES), jnp.float32),
            pltpu.VMEM_SHARED((n, LANES), jnp.float32),
        ],
        compiler_params=pltpu.CompilerParams(use_tc_tiling_on_sc=False),
        name="sc_degree_hist",
    )
    def hist(src_hbm, dst_hbm, z_hbm, ho_hbm, hi_hbm,
             srcv, dstv, srcv_t, dstv_t, ones_v, hist_o, hist_i):
        cid = lax.axis_index("c")
        sid = lax.axis_index("s")
        wid = sid * NC + cid

        @pl.loop(0, ECHUNK)
        def _(i):
            ones_v[i, :] = jnp.ones((LANES,), jnp.float32)

        rs = sid * rps
        pltpu.sync_copy(z_hbm.at[pl.ds(rs, rps)], hist_o.at[pl.ds(rs, rps)])
        pltpu.sync_copy(z_hbm.at[pl.ds(rs, rps)], hist_i.at[pl.ds(rs, rps)])
        plsc.subcore_barrier()

        ebase = wid * ew

        @pl.loop(0, full)
        def _(k):
            b = ebase + k * ECHUNK
            pltpu.sync_copy(src_hbm.at[pl.ds(b, ECHUNK)], srcv)
            pltpu.sync_copy(dst_hbm.at[pl.ds(b, ECHUNK)], dstv)
            pltpu.sync_copy(ones_v, hist_o.at[srcv], add=True)
            pltpu.sync_copy(ones_v, hist_i.at[dstv], add=True)

        @pl.loop(0, tail16)
        def _(t):
            b = ebase + full * ECHUNK + t * LANES
            pltpu.sync_copy(src_hbm.at[pl.ds(b, LANES)], srcv_t)
            pltpu.sync_copy(dst_hbm.at[pl.ds(b, LANES)], dstv_t)
            pltpu.sync_copy(ones_v.at[pl.ds(0, LANES)], hist_o.at[srcv_t], add=True)
            pltpu.sync_copy(ones_v.at[pl.ds(0, LANES)], hist_i.at[dstv_t], add=True)

        plsc.subcore_barrier()
        pltpu.sync_copy(hist_o.at[pl.ds(rs, rps)], ho_hbm.at[cid, pl.ds(rs, rps)])
        pltpu.sync_copy(hist_i.at[pl.ds(rs, rps)], hi_hbm.at[cid, pl.ds(rs, rps)])

    return hist(src, dst, z16)


def _make_prop(n, hid, e):
    ew = e // NW
    full = ew // ECHUNK
    tail16 = (ew - full * ECHUNK) // LANES
    rps = n // NS
    qn = hid // LANES
    mesh = plsc.VectorSubcoreMesh(core_axis_name="c", subcore_axis_name="s")

    @functools.partial(
        pl.kernel,
        out_type=jax.ShapeDtypeStruct((NC, n, hid), jnp.float32),
        mesh=mesh,
        scratch_types=[
            pltpu.VMEM((ECHUNK,), jnp.int32),
            pltpu.VMEM((ECHUNK,), jnp.int32),
            pltpu.VMEM((LANES,), jnp.int32),
            pltpu.VMEM((LANES,), jnp.int32),
            pltpu.VMEM((ECHUNK, hid), jnp.float32),
            pltpu.VMEM((LANES, hid), jnp.float32),
            pltpu.VMEM((ECHUNK,), jnp.float32),
            pltpu.VMEM((LANES,), jnp.float32),
            pltpu.VMEM_SHARED((n, hid), jnp.float32),
        ],
        compiler_params=pltpu.CompilerParams(use_tc_tiling_on_sc=False),
        name="sc_propagate",
    )
    def prop(hs_hbm, src_hbm, dst_hbm, w_hbm, z_hbm, p_hbm,
             srcv, dstv, srcv_t, dstv_t, rows, rows_t, w_vm, w_vm_t, agg):
        cid = lax.axis_index("c")
        sid = lax.axis_index("s")
        wid = sid * NC + cid

        rs = sid * rps
        pltpu.sync_copy(z_hbm.at[pl.ds(rs, rps)], agg.at[pl.ds(rs, rps)])
        plsc.subcore_barrier()

        ebase = wid * ew

        @pl.loop(0, full)
        def _(k):
            b = ebase + k * ECHUNK
            pltpu.sync_copy(src_hbm.at[pl.ds(b, ECHUNK)], srcv)
            pltpu.sync_copy(dst_hbm.at[pl.ds(b, ECHUNK)], dstv)
            pltpu.sync_copy(w_hbm.at[pl.ds(b, ECHUNK)], w_vm)
            pltpu.sync_copy(hs_hbm.at[srcv], rows)

            @pl.loop(0, ECHUNK // LANES)
            def _(j):
                wv = w_vm[pl.ds(j * LANES, LANES)]
                for l in range(LANES):
                    wl = wv.at[jnp.full((LANES,), l, jnp.int32)].get(
                        mode="promise_in_bounds")
                    i = j * LANES + l
                    for q in range(qn):
                        rows[i, pl.ds(q * LANES, LANES)] = (
                            rows[i, pl.ds(q * LANES, LANES)] * wl)

            pltpu.sync_copy(rows, agg.at[dstv], add=True)

        @pl.loop(0, tail16)
        def _(t):
            b = ebase + full * ECHUNK + t * LANES
            pltpu.sync_copy(src_hbm.at[pl.ds(b, LANES)], srcv_t)
            pltpu.sync_copy(dst_hbm.at[pl.ds(b, LANES)], dstv_t)
            pltpu.sync_copy(w_hbm.at[pl.ds(b, LANES)], w_vm_t)
            pltpu.sync_copy(hs_hbm.at[srcv_t], rows_t)

            wv = w_vm_t[...]
            for l in range(LANES):
                wl = wv.at[jnp.full((LANES,), l, jnp.int32)].get(
                    mode="promise_in_bounds")
                for q in range(qn):
                    rows_t[l, pl.ds(q * LANES, LANES)] = (
                        rows_t[l, pl.ds(q * LANES, LANES)] * wl)

            pltpu.sync_copy(rows_t, agg.at[dstv_t], add=True)

        plsc.subcore_barrier()
        pltpu.sync_copy(agg.at[pl.ds(rs, rps)], p_hbm.at[cid, pl.ds(rs, rps)])

    return prop


def kernel(x, edge_index, edge_weight, W_in, b_in, W_out, b_out):
    n0 = x.shape[0]
    hid = W_in.shape[1]
    e = edge_index.shape[1]
    src = edge_index[0]
    dst = edge_index[1]

    n = ((n0 + 8 * NS - 1) // (8 * NS)) * (8 * NS)
    x = jnp.pad(x, ((0, n - n0), (0, 0)))

    h0 = _tc_mlp_in(x, W_in, b_in.reshape(1, -1))
    z16 = jnp.zeros((n, LANES), jnp.float32)
    z64 = jnp.zeros((n, hid), jnp.float32)
    ho, hi = _sc_hist(src, dst, z16)
    hs, f0a, nins, nout = _tc_norm(h0, ho, hi)

    prop = _make_prop(n, hid, e)
    p = None
    for r in range(KHOP):
        p = prop(hs, src, dst, edge_weight, z64)
        if r < KHOP - 1:
            hs = _tc_combine(p, nins, nout, f0a)
    out = _tc_final(p, nins, f0a, W_out, b_out.reshape(1, -1))
    return out[:n0]


# speedup vs baseline: 5.2175x; 1.4403x over previous
import functools

import jax
import jax.numpy as jnp
from jax import lax
from jax.experimental import pallas as pl
from jax.experimental.pallas import tpu as pltpu
from jax.experimental.pallas import tpu_sc as plsc

ALPHA = 0.1
KHOP = 3
NC = 2
NS = 16
NW = NC * NS
ECHUNK = 128
LANES = 16


def _mlp_in_body(x_ref, w_ref, b_ref, o_ref):
    h = jnp.dot(x_ref[...], w_ref[...], preferred_element_type=jnp.float32)
    o_ref[...] = jnp.maximum(h + b_ref[...], 0.0)


def _norm_body(h0_ref, ho_ref, hi_ref, hs_ref, f0a_ref, nins_ref, nout_ref):
    deg_out = ho_ref[0, :, 0:1] + ho_ref[1, :, 0:1]
    deg_in = hi_ref[0, :, 0:1] + hi_ref[1, :, 0:1]
    no = jnp.where(deg_out > 0, lax.rsqrt(deg_out), 0.0)
    ni = jnp.where(deg_in > 0, lax.rsqrt(deg_in), 0.0)
    h0 = h0_ref[...]
    hs_ref[...] = h0 * no
    f0a_ref[...] = ALPHA * h0
    nins_ref[...] = (1.0 - ALPHA) * ni
    nout_ref[...] = no


def _combine_body(p_ref, nins_ref, nout_ref, f0a_ref, o_ref):
    h = (p_ref[0] + p_ref[1]) * nins_ref[...] + f0a_ref[...]
    o_ref[...] = h * nout_ref[...]


def _final_body(p_ref, nins_ref, f0a_ref, w_ref, b_ref, o_ref):
    h = (p_ref[0] + p_ref[1]) * nins_ref[...] + f0a_ref[...]
    o_ref[...] = jnp.dot(h, w_ref[...], preferred_element_type=jnp.float32) + b_ref[...]


def _tc_mlp_in(x, w, b):
    return pl.pallas_call(
        _mlp_in_body,
        out_shape=jax.ShapeDtypeStruct((x.shape[0], w.shape[1]), jnp.float32),
        name="tc_mlp_in",
    )(x, w, b)


def _tc_norm(h0, ho, hi):
    n, hid = h0.shape
    return pl.pallas_call(
        _norm_body,
        out_shape=[
            jax.ShapeDtypeStruct((n, hid), jnp.float32),
            jax.ShapeDtypeStruct((n, hid), jnp.float32),
            jax.ShapeDtypeStruct((n, 1), jnp.float32),
            jax.ShapeDtypeStruct((n, 1), jnp.float32),
        ],
        name="tc_norm",
    )(h0, ho, hi)


def _tc_combine(p, nins, nout, f0a):
    return pl.pallas_call(
        _combine_body,
        out_shape=jax.ShapeDtypeStruct(f0a.shape, jnp.float32),
        name="tc_combine",
    )(p, nins, nout, f0a)


def _tc_final(p, nins, f0a, w, b):
    n = f0a.shape[0]
    return pl.pallas_call(
        _final_body,
        out_shape=jax.ShapeDtypeStruct((n, w.shape[1]), jnp.float32),
        name="tc_final",
    )(p, nins, f0a, w, b)


def _sc_hist(src, dst, z16):
    n = z16.shape[0]
    e = src.shape[0]
    ew = e // NW
    full = ew // ECHUNK
    tail16 = (ew - full * ECHUNK) // LANES
    rps = n // NS
    mesh = plsc.VectorSubcoreMesh(core_axis_name="c", subcore_axis_name="s")
    out_t = jax.ShapeDtypeStruct((NC, n, LANES), jnp.float32)

    @functools.partial(
        pl.kernel,
        out_type=[out_t, out_t],
        mesh=mesh,
        scratch_types=[
            pltpu.VMEM((ECHUNK,), jnp.int32),
            pltpu.VMEM((ECHUNK,), jnp.int32),
            pltpu.VMEM((LANES,), jnp.int32),
            pltpu.VMEM((LANES,), jnp.int32),
            pltpu.VMEM((ECHUNK, LANES), jnp.float32),
            pltpu.VMEM_SHARED((n, LANES), jnp.float32),
            pltpu.VMEM_SHARED((n, LANES), jnp.float32),
        ],
        compiler_params=pltpu.CompilerParams(use_tc_tiling_on_sc=False),
        name="sc_degree_hist",
    )
    def hist(src_hbm, dst_hbm, z_hbm, ho_hbm, hi_hbm,
             srcv, dstv, srcv_t, dstv_t, ones_v, hist_o, hist_i):
        cid = lax.axis_index("c")
        sid = lax.axis_index("s")
        wid = sid * NC + cid

        @pl.loop(0, ECHUNK)
        def _(i):
            ones_v[i, :] = jnp.ones((LANES,), jnp.float32)

        rs = sid * rps
        pltpu.sync_copy(z_hbm.at[pl.ds(rs, rps)], hist_o.at[pl.ds(rs, rps)])
        pltpu.sync_copy(z_hbm.at[pl.ds(rs, rps)], hist_i.at[pl.ds(rs, rps)])
        plsc.subcore_barrier()

        ebase = wid * ew

        @pl.loop(0, full)
        def _(k):
            b = ebase + k * ECHUNK
            pltpu.sync_copy(src_hbm.at[pl.ds(b, ECHUNK)], srcv)
            pltpu.sync_copy(dst_hbm.at[pl.ds(b, ECHUNK)], dstv)
            pltpu.sync_copy(ones_v, hist_o.at[srcv], add=True)
            pltpu.sync_copy(ones_v, hist_i.at[dstv], add=True)

        @pl.loop(0, tail16)
        def _(t):
            b = ebase + full * ECHUNK + t * LANES
            pltpu.sync_copy(src_hbm.at[pl.ds(b, LANES)], srcv_t)
            pltpu.sync_copy(dst_hbm.at[pl.ds(b, LANES)], dstv_t)
            pltpu.sync_copy(ones_v.at[pl.ds(0, LANES)], hist_o.at[srcv_t], add=True)
            pltpu.sync_copy(ones_v.at[pl.ds(0, LANES)], hist_i.at[dstv_t], add=True)

        plsc.subcore_barrier()
        pltpu.sync_copy(hist_o.at[pl.ds(rs, rps)], ho_hbm.at[cid, pl.ds(rs, rps)])
        pltpu.sync_copy(hist_i.at[pl.ds(rs, rps)], hi_hbm.at[cid, pl.ds(rs, rps)])

    return hist(src, dst, z16)


def _make_prop(n, hid, e):
    ew = e // NW
    full = ew // ECHUNK
    tail16 = (ew - full * ECHUNK) // LANES
    rps = n // NS
    qn = hid // LANES
    mesh = plsc.VectorSubcoreMesh(core_axis_name="c", subcore_axis_name="s")

    @functools.partial(
        pl.kernel,
        out_type=jax.ShapeDtypeStruct((NC, n, hid), jnp.float32),
        mesh=mesh,
        scratch_types=[
            pltpu.VMEM((ECHUNK,), jnp.int32),
            pltpu.VMEM((ECHUNK,), jnp.int32),
            pltpu.VMEM((ECHUNK,), jnp.int32),
            pltpu.VMEM((ECHUNK,), jnp.int32),
            pltpu.VMEM((LANES,), jnp.int32),
            pltpu.VMEM((LANES,), jnp.int32),
            pltpu.VMEM((ECHUNK, hid), jnp.float32),
            pltpu.VMEM((ECHUNK, hid), jnp.float32),
            pltpu.VMEM((LANES, hid), jnp.float32),
            pltpu.VMEM((ECHUNK,), jnp.float32),
            pltpu.VMEM((ECHUNK,), jnp.float32),
            pltpu.VMEM((LANES,), jnp.float32),
            pltpu.VMEM_SHARED((n, hid), jnp.float32),
            pltpu.SemaphoreType.DMA,
            pltpu.SemaphoreType.DMA,
            pltpu.SemaphoreType.DMA,
            pltpu.SemaphoreType.DMA,
            pltpu.SemaphoreType.DMA,
            pltpu.SemaphoreType.DMA,
        ],
        compiler_params=pltpu.CompilerParams(use_tc_tiling_on_sc=False),
        name="sc_propagate",
    )
    def prop(hs_hbm, src_hbm, dst_hbm, w_hbm, z_hbm, p_hbm,
             srcv0, srcv1, dstv0, dstv1, srcv_t, dstv_t,
             rows0, rows1, rows_t, w_vm0, w_vm1, w_vm_t, agg,
             sem_i0, sem_i1, sem_g0, sem_g1, sem_d0, sem_d1):
        cid = lax.axis_index("c")
        sid = lax.axis_index("s")
        wid = sid * NC + cid

        rs = sid * rps
        pltpu.sync_copy(z_hbm.at[pl.ds(rs, rps)], agg.at[pl.ds(rs, rps)])
        plsc.subcore_barrier()

        ebase = wid * ew
        bufs = ((srcv0, dstv0, w_vm0, rows0, sem_i0, sem_g0, sem_d0),
                (srcv1, dstv1, w_vm1, rows1, sem_i1, sem_g1, sem_d1))

        def issue_idx(k, b):
            srcv, dstv, w_vm, _, sem_i, _, sem_d = bufs[b]
            base = ebase + k * ECHUNK
            pltpu.async_copy(src_hbm.at[pl.ds(base, ECHUNK)], srcv, sem_i)
            pltpu.async_copy(dst_hbm.at[pl.ds(base, ECHUNK)], dstv, sem_d)
            pltpu.async_copy(w_hbm.at[pl.ds(base, ECHUNK)], w_vm, sem_d)

        def wait_src(k, b):
            srcv, _, _, _, sem_i, _, _ = bufs[b]
            base = ebase + k * ECHUNK
            pltpu.make_async_copy(src_hbm.at[pl.ds(base, ECHUNK)], srcv, sem_i).wait()

        def wait_dw(k, b):
            _, dstv, w_vm, _, _, _, sem_d = bufs[b]
            base = ebase + k * ECHUNK
            pltpu.make_async_copy(dst_hbm.at[pl.ds(base, ECHUNK)], dstv, sem_d).wait()
            pltpu.make_async_copy(w_hbm.at[pl.ds(base, ECHUNK)], w_vm, sem_d).wait()

        def issue_gather(b):
            srcv, _, _, rows, _, sem_g, _ = bufs[b]
            pltpu.async_copy(hs_hbm.at[srcv], rows, sem_g)

        def wait_gather(b):
            srcv, _, _, rows, _, sem_g, _ = bufs[b]
            pltpu.make_async_copy(hs_hbm.at[srcv], rows, sem_g).wait()

        def scale_scatter(b):
            _, dstv, w_vm, rows, _, _, _ = bufs[b]

            @pl.loop(0, ECHUNK // LANES)
            def _(j):
                wv = w_vm[pl.ds(j * LANES, LANES)]
                for l in range(LANES):
                    wl = wv.at[jnp.full((LANES,), l, jnp.int32)].get(
                        mode="promise_in_bounds")
                    i = j * LANES + l
                    for q in range(qn):
                        rows[i, pl.ds(q * LANES, LANES)] = (
                            rows[i, pl.ds(q * LANES, LANES)] * wl)

            pltpu.sync_copy(rows, agg.at[dstv], add=True)

        issue_idx(0, 0)
        wait_src(0, 0)
        issue_gather(0)
        issue_idx(1, 1)

        @pl.loop(0, (full - 2) // 2)
        def _(t):
            k = 2 * t
            wait_src(k + 1, 1)
            issue_gather(1)
            wait_gather(0)
            wait_dw(k, 0)
            scale_scatter(0)
            issue_idx(k + 2, 0)
            wait_src(k + 2, 0)
            issue_gather(0)
            wait_gather(1)
            wait_dw(k + 1, 1)
            scale_scatter(1)
            issue_idx(k + 3, 1)

        wait_src(full - 1, 1)
        issue_gather(1)
        wait_gather(0)
        wait_dw(full - 2, 0)
        scale_scatter(0)
        wait_gather(1)
        wait_dw(full - 1, 1)
        scale_scatter(1)

        @pl.loop(0, tail16)
        def _(t):
            b = ebase + full * ECHUNK + t * LANES
            pltpu.sync_copy(src_hbm.at[pl.ds(b, LANES)], srcv_t)
            pltpu.sync_copy(dst_hbm.at[pl.ds(b, LANES)], dstv_t)
            pltpu.sync_copy(w_hbm.at[pl.ds(b, LANES)], w_vm_t)
            pltpu.sync_copy(hs_hbm.at[srcv_t], rows_t)

            wv = w_vm_t[...]
            for l in range(LANES):
                wl = wv.at[jnp.full((LANES,), l, jnp.int32)].get(
                    mode="promise_in_bounds")
                for q in range(qn):
                    rows_t[l, pl.ds(q * LANES, LANES)] = (
                        rows_t[l, pl.ds(q * LANES, LANES)] * wl)

            pltpu.sync_copy(rows_t, agg.at[dstv_t], add=True)

        plsc.subcore_barrier()
        pltpu.sync_copy(agg.at[pl.ds(rs, rps)], p_hbm.at[cid, pl.ds(rs, rps)])

    return prop


def kernel(x, edge_index, edge_weight, W_in, b_in, W_out, b_out):
    n0 = x.shape[0]
    hid = W_in.shape[1]
    e = edge_index.shape[1]
    src = edge_index[0]
    dst = edge_index[1]

    n = ((n0 + 8 * NS - 1) // (8 * NS)) * (8 * NS)
    x = jnp.pad(x, ((0, n - n0), (0, 0)))

    h0 = _tc_mlp_in(x, W_in, b_in.reshape(1, -1))
    z16 = jnp.zeros((n, LANES), jnp.float32)
    z64 = jnp.zeros((n, hid), jnp.float32)
    ho, hi = _sc_hist(src, dst, z16)
    hs, f0a, nins, nout = _tc_norm(h0, ho, hi)

    prop = _make_prop(n, hid, e)
    p = None
    for r in range(KHOP):
        p = prop(hs, src, dst, edge_weight, z64)
        if r < KHOP - 1:
            hs = _tc_combine(p, nins, nout, f0a)
    out = _tc_final(p, nins, f0a, W_out, b_out.reshape(1, -1))
    return out[:n0]
